# baseline (speedup 1.0000x reference)
import functools

import jax
import jax.numpy as jnp
import numpy as np
from jax import lax
from jax.experimental import pallas as pl
from jax.experimental.pallas import tpu as pltpu
from jax.experimental.pallas import tpu_sc as plsc

N = 8388608
TOTAL_BINS = 64 * 64 * 64

NUM_CORES = 1
NUM_SUBCORES = 16
NW = NUM_CORES * NUM_SUBCORES
PTS_PER_W = N // NW
CHUNK = 256
NCH = PTS_PER_W // CHUNK
VL = 16
BINS_PER_TILE = TOTAL_BINS // NUM_SUBCORES

SCALE = np.float32(64.0 / 6.0)
BIAS = np.float32(32.0 + 1024.0)


def _sc_body(vals_hbm, w_hbm, hist_out, oob_out,
             vals_v0, vals_v1, w_v0, w_v1, idx_v0, idx_v1, wv_v0, wv_v1,
             oob_v, hist_sp, sem0, sem1):
    cid = lax.axis_index("c")
    sid = lax.axis_index("s")
    wid = sid * NUM_CORES + cid
    vals_b = (vals_v0, vals_v1)
    w_b = (w_v0, w_v1)
    idx_b = (idx_v0, idx_v1)
    wv_b = (wv_v0, wv_v1)
    sem_b = (sem0, sem1)

    zeros16 = jnp.zeros((VL,), jnp.float32)
    lanes = lax.iota(jnp.int32, VL)

    def zero_body(k, _):
        wv_v0[pl.ds(k * VL, VL)] = zeros16
        return 0

    lax.fori_loop(0, CHUNK // VL, zero_body, 0)
    for q in range(BINS_PER_TILE // CHUNK):
        pltpu.sync_copy(
            wv_v0, hist_sp.at[pl.ds(sid * BINS_PER_TILE + q * CHUNK, CHUNK)]
        )
    plsc.subcore_barrier()

    dim0 = jnp.zeros((VL,), jnp.int32)
    dim1 = dim0 + 1
    dim2 = dim0 + 2
    spread0 = (lanes + wid * 8191) & (TOTAL_BINS - 1)
    w_base = wid * PTS_PER_W

    def fire(win, b):
        base = w_base + jnp.minimum(win, NCH - 1) * CHUNK
        pltpu.async_copy(vals_hbm.at[pl.ds(base, CHUNK)], vals_b[b], sem_b[b])
        pltpu.async_copy(w_hbm.at[pl.ds(base, CHUNK)], w_b[b], sem_b[b])

    def drain(b):
        pltpu.make_async_copy(
            vals_hbm.at[pl.ds(w_base, CHUNK)], vals_b[b], sem_b[b]
        ).wait()
        pltpu.make_async_copy(
            w_hbm.at[pl.ds(w_base, CHUNK)], w_b[b], sem_b[b]
        ).wait()

    def window(win, b, oob_acc):
        vals_v, w_v, idx_v, wv_v = vals_b[b], w_b[b], idx_b[b], wv_b[b]
        spread_base = (spread0 + win * CHUNK) & (TOTAL_BINS - 1)

        for j in range(CHUNK // VL):
            pt = lanes + j * VL
            vx = plsc.load_gather(vals_v, [pt, dim0])
            vy = plsc.load_gather(vals_v, [pt, dim1])
            vz = plsc.load_gather(vals_v, [pt, dim2])
            bx = (vx * SCALE + BIAS).astype(jnp.int32) - 1024
            by = (vy * SCALE + BIAS).astype(jnp.int32) - 1024
            bz = (vz * SCALE + BIAS).astype(jnp.int32) - 1024
            okx = plsc.bitcast(bx, jnp.uint32) < 64
            oky = plsc.bitcast(by, jnp.uint32) < 64
            okz = plsc.bitcast(bz, jnp.uint32) < 64
            ok = okx & oky & okz
            flat = (bx << 12) + (by << 6) + bz
            spread = (spread_base + j * VL) & (TOTAL_BINS - 1)
            flat = jnp.where(ok, flat, spread)
            w = w_v[pl.ds(j * VL, VL)]
            wv = jnp.where(ok, w, jnp.float32(0.0))
            idx_v[pl.ds(j * VL, VL)] = flat
            wv_v[pl.ds(j * VL, VL)] = wv
            oob_acc = oob_acc + (w - wv)
        pltpu.sync_copy(wv_v, hist_sp.at[idx_v], add=True)
        return oob_acc

    fire(0, 0)

    def pair_body(g2, oob_acc):
        win = g2 * 2
        fire(win + 1, 1)
        drain(0)
        oob_acc = window(win, 0, oob_acc)
        fire(win + 2, 0)
        drain(1)
        oob_acc = window(win + 1, 1, oob_acc)
        return oob_acc

    oob_acc = lax.fori_loop(0, NCH // 2, pair_body,
                            jnp.zeros((VL,), jnp.float32))
    drain(0)
    plsc.subcore_barrier()

    pltpu.sync_copy(
        hist_sp.at[pl.ds(sid * BINS_PER_TILE, BINS_PER_TILE)],
        hist_out.at[cid, pl.ds(sid * BINS_PER_TILE, BINS_PER_TILE)],
    )
    oob_v[...] = oob_acc
    pltpu.sync_copy(oob_v, oob_out.at[cid, sid])


@functools.cache
def _build_sc_hist():
    return pl.kernel(
        _sc_body,
        out_type=(
            jax.ShapeDtypeStruct((NUM_CORES, TOTAL_BINS), jnp.float32),
            jax.ShapeDtypeStruct((NUM_CORES, NUM_SUBCORES, VL), jnp.float32),
        ),
        mesh=plsc.VectorSubcoreMesh(
            core_axis_name="c", subcore_axis_name="s",
            num_cores=NUM_CORES, num_subcores=NUM_SUBCORES,
        ),
        scratch_types=[
            pltpu.VMEM((CHUNK, 3), jnp.float32),
            pltpu.VMEM((CHUNK, 3), jnp.float32),
            pltpu.VMEM((CHUNK,), jnp.float32),
            pltpu.VMEM((CHUNK,), jnp.float32),
            pltpu.VMEM((CHUNK,), jnp.int32),
            pltpu.VMEM((CHUNK,), jnp.int32),
            pltpu.VMEM((CHUNK,), jnp.float32),
            pltpu.VMEM((CHUNK,), jnp.float32),
            pltpu.VMEM((VL,), jnp.float32),
            pltpu.VMEM_SHARED((TOTAL_BINS,), jnp.float32),
            pltpu.SemaphoreType.DMA,
            pltpu.SemaphoreType.DMA,
        ],
        compiler_params=pltpu.CompilerParams(needs_layout_passes=False),
    )


def _combine_body(hp_ref, oob_ref, hist_ref, oob_out_ref):
    hist_ref[...] = hp_ref[0]
    oob_out_ref[...] = jnp.sum(oob_ref[...])[None, None]


def kernel(values, weights):
    hist_parts, oob_parts = _build_sc_hist()(values, weights)
    hist2, oob11 = pl.pallas_call(
        _combine_body,
        out_shape=(
            jax.ShapeDtypeStruct((TOTAL_BINS // 128, 128), jnp.float32),
            jax.ShapeDtypeStruct((1, 1), jnp.float32),
        ),
    )(hist_parts.reshape(NUM_CORES, TOTAL_BINS // 128, 128),
      oob_parts.reshape(2, 128))
    return hist2.reshape(TOTAL_BINS), oob11[0, 0]


# speedup vs baseline: 18.3286x; 18.3286x over previous
import functools

import jax
import jax.numpy as jnp
import numpy as np
from jax import lax
from jax.experimental import pallas as pl
from jax.experimental.pallas import tpu as pltpu
from jax.experimental.pallas import tpu_sc as plsc

N = 8388608
TOTAL_BINS = 64 * 64 * 64
NROWS = N // 128

NUM_CORES = 2
NUM_SUBCORES = 16
NW = NUM_CORES * NUM_SUBCORES
PTS_PER_W = N // NW
CHUNK = 1024
ROWS_PER_W = CHUNK // 128
NCH = PTS_PER_W // CHUNK
VL = 16
BINS_PER_TILE = TOTAL_BINS // NUM_SUBCORES

SCALE = np.float32(64.0 / 6.0)
BIAS = np.float32(32.0 + 1024.0)


def _sc_body(vals_hbm, w_hbm, hist_out, oob_out,
             x_v0, x_v1, y_v0, y_v1, z_v0, z_v1, w_v0, w_v1,
             idx_v0, idx_v1, wv_v0, wv_v1, oob_v, hist_sp, sem0, sem1):
    cid = lax.axis_index("c")
    sid = lax.axis_index("s")
    wid = sid * NUM_CORES + cid
    x_b = (x_v0, x_v1)
    y_b = (y_v0, y_v1)
    z_b = (z_v0, z_v1)
    w_b = (w_v0, w_v1)
    idx_b = (idx_v0, idx_v1)
    wv_b = (wv_v0, wv_v1)
    sem_b = (sem0, sem1)

    zeros16 = jnp.zeros((VL,), jnp.float32)
    lanes = lax.iota(jnp.int32, VL)

    def zero_body(k, _):
        wv_v0[pl.ds(k * VL, VL)] = zeros16
        return 0

    lax.fori_loop(0, CHUNK // VL, zero_body, 0)
    for q in range(BINS_PER_TILE // CHUNK):
        pltpu.sync_copy(
            wv_v0, hist_sp.at[pl.ds(sid * BINS_PER_TILE + q * CHUNK, CHUNK)]
        )
    plsc.subcore_barrier()

    spread0 = (lanes + wid * 8191) & (TOTAL_BINS - 1)
    row_base = wid * (PTS_PER_W // 128)
    w_base = wid * PTS_PER_W

    def fire(win, b):
        wc = jnp.minimum(win, NCH - 1)
        r0 = row_base + wc * ROWS_PER_W
        pltpu.async_copy(vals_hbm.at[0, pl.ds(r0, ROWS_PER_W)], x_b[b], sem_b[b])
        pltpu.async_copy(vals_hbm.at[1, pl.ds(r0, ROWS_PER_W)], y_b[b], sem_b[b])
        pltpu.async_copy(vals_hbm.at[2, pl.ds(r0, ROWS_PER_W)], z_b[b], sem_b[b])
        pltpu.async_copy(w_hbm.at[pl.ds(w_base + wc * CHUNK, CHUNK)],
                         w_b[b], sem_b[b])

    def drain(b):
        for dst in (x_b[b], y_b[b], z_b[b]):
            pltpu.make_async_copy(
                vals_hbm.at[0, pl.ds(row_base, ROWS_PER_W)], dst, sem_b[b]
            ).wait()
        pltpu.make_async_copy(
            w_hbm.at[pl.ds(w_base, CHUNK)], w_b[b], sem_b[b]
        ).wait()

    def window(win, b, oob_acc):
        x_v, y_v, z_v = x_b[b], y_b[b], z_b[b]
        w_v, idx_v, wv_v = w_b[b], idx_b[b], wv_b[b]
        spread_base = (spread0 + win * CHUNK) & (TOTAL_BINS - 1)

        for j in range(CHUNK // VL):
            r, c = j // 8, (j % 8) * VL
            vx = x_v[r, pl.ds(c, VL)]
            vy = y_v[r, pl.ds(c, VL)]
            vz = z_v[r, pl.ds(c, VL)]
            bx = (vx * SCALE + BIAS).astype(jnp.int32) - 1024
            by = (vy * SCALE + BIAS).astype(jnp.int32) - 1024
            bz = (vz * SCALE + BIAS).astype(jnp.int32) - 1024
            okx = plsc.bitcast(bx, jnp.uint32) < 64
            oky = plsc.bitcast(by, jnp.uint32) < 64
            okz = plsc.bitcast(bz, jnp.uint32) < 64
            ok = okx & oky & okz
            flat = (bx << 12) + (by << 6) + bz
            spread = (spread_base + j * VL) & (TOTAL_BINS - 1)
            flat = jnp.where(ok, flat, spread)
            w = w_v[pl.ds(j * VL, VL)]
            wv = jnp.where(ok, w, jnp.float32(0.0))
            idx_v[pl.ds(j * VL, VL)] = flat
            wv_v[pl.ds(j * VL, VL)] = wv
            oob_acc = oob_acc + (w - wv)
        pltpu.sync_copy(wv_v, hist_sp.at[idx_v], add=True)
        return oob_acc

    fire(0, 0)

    def pair_body(g2, oob_acc):
        win = g2 * 2
        fire(win + 1, 1)
        drain(0)
        oob_acc = window(win, 0, oob_acc)
        fire(win + 2, 0)
        drain(1)
        oob_acc = window(win + 1, 1, oob_acc)
        return oob_acc

    oob_acc = lax.fori_loop(0, NCH // 2, pair_body,
                            jnp.zeros((VL,), jnp.float32))
    drain(0)
    plsc.subcore_barrier()

    pltpu.sync_copy(
        hist_sp.at[pl.ds(sid * BINS_PER_TILE, BINS_PER_TILE)],
        hist_out.at[cid, pl.ds(sid * BINS_PER_TILE, BINS_PER_TILE)],
    )
    oob_v[...] = oob_acc
    pltpu.sync_copy(oob_v, oob_out.at[cid, sid])


@functools.cache
def _build_sc_hist():
    return pl.kernel(
        _sc_body,
        out_type=(
            jax.ShapeDtypeStruct((NUM_CORES, TOTAL_BINS), jnp.float32),
            jax.ShapeDtypeStruct((NUM_CORES, NUM_SUBCORES, VL), jnp.float32),
        ),
        mesh=plsc.VectorSubcoreMesh(
            core_axis_name="c", subcore_axis_name="s",
            num_cores=NUM_CORES, num_subcores=NUM_SUBCORES,
        ),
        scratch_types=[
            pltpu.VMEM((ROWS_PER_W, 128), jnp.float32),
            pltpu.VMEM((ROWS_PER_W, 128), jnp.float32),
            pltpu.VMEM((ROWS_PER_W, 128), jnp.float32),
            pltpu.VMEM((ROWS_PER_W, 128), jnp.float32),
            pltpu.VMEM((ROWS_PER_W, 128), jnp.float32),
            pltpu.VMEM((ROWS_PER_W, 128), jnp.float32),
            pltpu.VMEM((CHUNK,), jnp.float32),
            pltpu.VMEM((CHUNK,), jnp.float32),
            pltpu.VMEM((CHUNK,), jnp.int32),
            pltpu.VMEM((CHUNK,), jnp.int32),
            pltpu.VMEM((CHUNK,), jnp.float32),
            pltpu.VMEM((CHUNK,), jnp.float32),
            pltpu.VMEM((VL,), jnp.float32),
            pltpu.VMEM_SHARED((TOTAL_BINS,), jnp.float32),
            pltpu.SemaphoreType.DMA,
            pltpu.SemaphoreType.DMA,
        ],
        compiler_params=pltpu.CompilerParams(needs_layout_passes=False),
    )


def _combine_body(hp_ref, oob_ref, hist_ref, oob_out_ref):
    hist_ref[...] = hp_ref[0] + hp_ref[1]
    oob_out_ref[...] = jnp.sum(oob_ref[...])[None, None]


def kernel(values, weights):
    vals_t = jnp.transpose(values.reshape(NROWS, 128, 3), (2, 0, 1))
    hist_parts, oob_parts = _build_sc_hist()(vals_t, weights)
    hist2, oob11 = pl.pallas_call(
        _combine_body,
        out_shape=(
            jax.ShapeDtypeStruct((TOTAL_BINS // 128, 128), jnp.float32),
            jax.ShapeDtypeStruct((1, 1), jnp.float32),
        ),
    )(hist_parts.reshape(NUM_CORES, TOTAL_BINS // 128, 128),
      oob_parts.reshape(4, 128))
    return hist2.reshape(TOTAL_BINS), oob11[0, 0]


# speedup vs baseline: 19.4624x; 1.0619x over previous
import functools

import jax
import jax.numpy as jnp
import numpy as np
from jax import lax
from jax.experimental import pallas as pl
from jax.experimental.pallas import tpu as pltpu
from jax.experimental.pallas import tpu_sc as plsc

N = 8388608
TOTAL_BINS = 64 * 64 * 64
NROWS = N // 128

NUM_CORES = 2
NUM_SUBCORES = 16
NW = NUM_CORES * NUM_SUBCORES
PTS_PER_W = N // NW
CHUNK = 2048
ROWS_PER_W = CHUNK // 128
NCH = PTS_PER_W // CHUNK
VL = 16
BINS_PER_TILE = TOTAL_BINS // NUM_SUBCORES

SCALE = np.float32(64.0 / 6.0)
BIAS = np.float32(32.0 + 1024.0)


def _sc_body(vals_hbm, w_hbm, hist_out, oob_out,
             x_v0, x_v1, y_v0, y_v1, z_v0, z_v1, w_v0, w_v1,
             idx_v0, idx_v1, wv_v0, wv_v1, oob_v, hist_sp, sem0, sem1):
    cid = lax.axis_index("c")
    sid = lax.axis_index("s")
    wid = sid * NUM_CORES + cid
    x_b = (x_v0, x_v1)
    y_b = (y_v0, y_v1)
    z_b = (z_v0, z_v1)
    w_b = (w_v0, w_v1)
    idx_b = (idx_v0, idx_v1)
    wv_b = (wv_v0, wv_v1)
    sem_b = (sem0, sem1)

    zeros16 = jnp.zeros((VL,), jnp.float32)
    lanes = lax.iota(jnp.int32, VL)

    def zero_body(k, _):
        wv_v0[pl.ds(k * VL, VL)] = zeros16
        return 0

    lax.fori_loop(0, CHUNK // VL, zero_body, 0)
    for q in range(BINS_PER_TILE // CHUNK):
        pltpu.sync_copy(
            wv_v0, hist_sp.at[pl.ds(sid * BINS_PER_TILE + q * CHUNK, CHUNK)]
        )
    plsc.subcore_barrier()

    spread0 = (lanes + wid * 8191) & (TOTAL_BINS - 1)
    row_base = wid * (PTS_PER_W // 128)
    w_base = wid * PTS_PER_W

    def fire(win, b):
        wc = jnp.minimum(win, NCH - 1)
        r0 = row_base + wc * ROWS_PER_W
        pltpu.async_copy(vals_hbm.at[0, pl.ds(r0, ROWS_PER_W)], x_b[b], sem_b[b])
        pltpu.async_copy(vals_hbm.at[1, pl.ds(r0, ROWS_PER_W)], y_b[b], sem_b[b])
        pltpu.async_copy(vals_hbm.at[2, pl.ds(r0, ROWS_PER_W)], z_b[b], sem_b[b])
        pltpu.async_copy(w_hbm.at[pl.ds(w_base + wc * CHUNK, CHUNK)],
                         w_b[b], sem_b[b])

    def drain(b):
        for dst in (x_b[b], y_b[b], z_b[b]):
            pltpu.make_async_copy(
                vals_hbm.at[0, pl.ds(row_base, ROWS_PER_W)], dst, sem_b[b]
            ).wait()
        pltpu.make_async_copy(
            w_hbm.at[pl.ds(w_base, CHUNK)], w_b[b], sem_b[b]
        ).wait()

    def window(win, b, oob_acc):
        x_v, y_v, z_v = x_b[b], y_b[b], z_b[b]
        w_v, idx_v, wv_v = w_b[b], idx_b[b], wv_b[b]
        spread_base = (spread0 + win * CHUNK) & (TOTAL_BINS - 1)

        def blk(jb, acc):
          for jj in range(32):
            j = jb * 32 + jj
            r, c = j // 8, (j % 8) * VL
            vx = x_v[r, pl.ds(c, VL)]
            vy = y_v[r, pl.ds(c, VL)]
            vz = z_v[r, pl.ds(c, VL)]
            bx = (vx * SCALE + BIAS).astype(jnp.int32) - 1024
            by = (vy * SCALE + BIAS).astype(jnp.int32) - 1024
            bz = (vz * SCALE + BIAS).astype(jnp.int32) - 1024
            okx = plsc.bitcast(bx, jnp.uint32) < 64
            oky = plsc.bitcast(by, jnp.uint32) < 64
            okz = plsc.bitcast(bz, jnp.uint32) < 64
            ok = okx & oky & okz
            flat = (bx << 12) + (by << 6) + bz
            spread = (spread_base + j * VL) & (TOTAL_BINS - 1)
            flat = jnp.where(ok, flat, spread)
            w = w_v[pl.ds(j * VL, VL)]
            wv = jnp.where(ok, w, jnp.float32(0.0))
            idx_v[pl.ds(j * VL, VL)] = flat
            wv_v[pl.ds(j * VL, VL)] = wv
            acc = acc + (w - wv)
          return acc

        oob_acc = lax.fori_loop(0, CHUNK // VL // 32, blk, oob_acc)
        pltpu.sync_copy(wv_v, hist_sp.at[idx_v], add=True)
        return oob_acc

    fire(0, 0)

    def pair_body(g2, oob_acc):
        win = g2 * 2
        fire(win + 1, 1)
        drain(0)
        oob_acc = window(win, 0, oob_acc)
        fire(win + 2, 0)
        drain(1)
        oob_acc = window(win + 1, 1, oob_acc)
        return oob_acc

    oob_acc = lax.fori_loop(0, NCH // 2, pair_body,
                            jnp.zeros((VL,), jnp.float32))
    drain(0)
    plsc.subcore_barrier()

    pltpu.sync_copy(
        hist_sp.at[pl.ds(sid * BINS_PER_TILE, BINS_PER_TILE)],
        hist_out.at[cid, pl.ds(sid * BINS_PER_TILE, BINS_PER_TILE)],
    )
    oob_v[...] = oob_acc
    pltpu.sync_copy(oob_v, oob_out.at[cid, sid])


@functools.cache
def _build_sc_hist():
    return pl.kernel(
        _sc_body,
        out_type=(
            jax.ShapeDtypeStruct((NUM_CORES, TOTAL_BINS), jnp.float32),
            jax.ShapeDtypeStruct((NUM_CORES, NUM_SUBCORES, VL), jnp.float32),
        ),
        mesh=plsc.VectorSubcoreMesh(
            core_axis_name="c", subcore_axis_name="s",
            num_cores=NUM_CORES, num_subcores=NUM_SUBCORES,
        ),
        scratch_types=[
            pltpu.VMEM((ROWS_PER_W, 128), jnp.float32),
            pltpu.VMEM((ROWS_PER_W, 128), jnp.float32),
            pltpu.VMEM((ROWS_PER_W, 128), jnp.float32),
            pltpu.VMEM((ROWS_PER_W, 128), jnp.float32),
            pltpu.VMEM((ROWS_PER_W, 128), jnp.float32),
            pltpu.VMEM((ROWS_PER_W, 128), jnp.float32),
            pltpu.VMEM((CHUNK,), jnp.float32),
            pltpu.VMEM((CHUNK,), jnp.float32),
            pltpu.VMEM((CHUNK,), jnp.int32),
            pltpu.VMEM((CHUNK,), jnp.int32),
            pltpu.VMEM((CHUNK,), jnp.float32),
            pltpu.VMEM((CHUNK,), jnp.float32),
            pltpu.VMEM((VL,), jnp.float32),
            pltpu.VMEM_SHARED((TOTAL_BINS,), jnp.float32),
            pltpu.SemaphoreType.DMA,
            pltpu.SemaphoreType.DMA,
        ],
        compiler_params=pltpu.CompilerParams(needs_layout_passes=False),
    )


def _combine_body(hp_ref, oob_ref, hist_ref, oob_out_ref):
    hist_ref[...] = hp_ref[0] + hp_ref[1]
    oob_out_ref[...] = jnp.sum(oob_ref[...])[None, None]


def kernel(values, weights):
    vals_t = jnp.transpose(values.reshape(NROWS, 128, 3), (2, 0, 1))
    hist_parts, oob_parts = _build_sc_hist()(vals_t, weights)
    hist2, oob11 = pl.pallas_call(
        _combine_body,
        out_shape=(
            jax.ShapeDtypeStruct((TOTAL_BINS // 128, 128), jnp.float32),
            jax.ShapeDtypeStruct((1, 1), jnp.float32),
        ),
    )(hist_parts.reshape(NUM_CORES, TOTAL_BINS // 128, 128),
      oob_parts.reshape(4, 128))
    return hist2.reshape(TOTAL_BINS), oob11[0, 0]


# speedup vs baseline: 21.0883x; 1.0835x over previous
import functools

import jax
import jax.numpy as jnp
import numpy as np
from jax import lax
from jax.experimental import pallas as pl
from jax.experimental.pallas import tpu as pltpu
from jax.experimental.pallas import tpu_sc as plsc

N = 8388608
TOTAL_BINS = 64 * 64 * 64
NROWS = N // 128

NUM_CORES = 2
NUM_SUBCORES = 16
NW = NUM_CORES * NUM_SUBCORES
PTS_PER_W = N // NW
CHUNK = 2048
ROWS_PER_W = CHUNK // 128
NCH = PTS_PER_W // CHUNK
VL = 16
BINS_PER_TILE = TOTAL_BINS // NUM_SUBCORES

SCALE = np.float32(64.0 / 6.0)
BIAS = np.float32(32.0 + 1024.0)


def _sc_body(vals_hbm, w_hbm, hist_out, oob_out,
             x_v0, x_v1, y_v0, y_v1, z_v0, z_v1, w_v0, w_v1,
             idx_v0, idx_v1, wv_v0, wv_v1, oob_v, hist_sp, sem0, sem1,
             ssem0, ssem1):
    cid = lax.axis_index("c")
    sid = lax.axis_index("s")
    wid = sid * NUM_CORES + cid
    x_b = (x_v0, x_v1)
    y_b = (y_v0, y_v1)
    z_b = (z_v0, z_v1)
    w_b = (w_v0, w_v1)
    idx_b = (idx_v0, idx_v1)
    wv_b = (wv_v0, wv_v1)
    sem_b = (sem0, sem1)
    ssem_b = (ssem0, ssem1)

    zeros16 = jnp.zeros((VL,), jnp.float32)
    lanes = lax.iota(jnp.int32, VL)

    def zero_body(k, _):
        wv_v0[pl.ds(k * VL, VL)] = zeros16
        return 0

    lax.fori_loop(0, CHUNK // VL, zero_body, 0)
    for q in range(BINS_PER_TILE // CHUNK):
        pltpu.sync_copy(
            wv_v0, hist_sp.at[pl.ds(sid * BINS_PER_TILE + q * CHUNK, CHUNK)]
        )
    plsc.subcore_barrier()

    spread0 = (lanes + wid * 8191) & (TOTAL_BINS - 1)
    row_base = wid * (PTS_PER_W // 128)
    w_base = wid * PTS_PER_W

    def fire(win, b):
        wc = jnp.minimum(win, NCH - 1)
        r0 = row_base + wc * ROWS_PER_W
        pltpu.async_copy(vals_hbm.at[0, pl.ds(r0, ROWS_PER_W)], x_b[b], sem_b[b])
        pltpu.async_copy(vals_hbm.at[1, pl.ds(r0, ROWS_PER_W)], y_b[b], sem_b[b])
        pltpu.async_copy(vals_hbm.at[2, pl.ds(r0, ROWS_PER_W)], z_b[b], sem_b[b])
        pltpu.async_copy(w_hbm.at[pl.ds(w_base + wc * CHUNK, CHUNK)],
                         w_b[b], sem_b[b])

    def drain(b):
        for dst in (x_b[b], y_b[b], z_b[b]):
            pltpu.make_async_copy(
                vals_hbm.at[0, pl.ds(row_base, ROWS_PER_W)], dst, sem_b[b]
            ).wait()
        pltpu.make_async_copy(
            w_hbm.at[pl.ds(w_base, CHUNK)], w_b[b], sem_b[b]
        ).wait()

    def window(win, b, oob_acc, wait_scatter=True):
        x_v, y_v, z_v = x_b[b], y_b[b], z_b[b]
        w_v, idx_v, wv_v = w_b[b], idx_b[b], wv_b[b]
        if wait_scatter:
            pltpu.make_async_copy(wv_v, hist_sp.at[idx_v], ssem_b[b]).wait()
        spread_base = (spread0 + win * CHUNK) & (TOTAL_BINS - 1)

        def blk(jb, acc):
          for jj in range(32):
            j = jb * 32 + jj
            r, c = j // 8, (j % 8) * VL
            vx = x_v[r, pl.ds(c, VL)]
            vy = y_v[r, pl.ds(c, VL)]
            vz = z_v[r, pl.ds(c, VL)]
            bx = (vx * SCALE + BIAS).astype(jnp.int32) - 1024
            by = (vy * SCALE + BIAS).astype(jnp.int32) - 1024
            bz = (vz * SCALE + BIAS).astype(jnp.int32) - 1024
            okx = plsc.bitcast(bx, jnp.uint32) < 64
            oky = plsc.bitcast(by, jnp.uint32) < 64
            okz = plsc.bitcast(bz, jnp.uint32) < 64
            ok = okx & oky & okz
            flat = (bx << 12) + (by << 6) + bz
            spread = (spread_base + j * VL) & (TOTAL_BINS - 1)
            flat = jnp.where(ok, flat, spread)
            w = w_v[pl.ds(j * VL, VL)]
            wv = jnp.where(ok, w, jnp.float32(0.0))
            idx_v[pl.ds(j * VL, VL)] = flat
            wv_v[pl.ds(j * VL, VL)] = wv
            acc = acc + (w - wv)
          return acc

        oob_acc = lax.fori_loop(0, CHUNK // VL // 32, blk, oob_acc)
        pltpu.async_copy(wv_v, hist_sp.at[idx_v], ssem_b[b], add=True)
        return oob_acc

    fire(0, 0)
    oob_acc = jnp.zeros((VL,), jnp.float32)
    fire(1, 1)
    drain(0)
    oob_acc = window(0, 0, oob_acc, wait_scatter=False)
    fire(2, 0)
    drain(1)
    oob_acc = window(1, 1, oob_acc, wait_scatter=False)

    def pair_body(g2, oob_acc):
        win = g2 * 2
        fire(win + 1, 1)
        drain(0)
        oob_acc = window(win, 0, oob_acc)
        fire(win + 2, 0)
        drain(1)
        oob_acc = window(win + 1, 1, oob_acc)
        return oob_acc

    oob_acc = lax.fori_loop(1, NCH // 2, pair_body, oob_acc)
    drain(0)
    for b in (0, 1):
        pltpu.make_async_copy(wv_b[b], hist_sp.at[idx_b[b]], ssem_b[b]).wait()
    plsc.subcore_barrier()

    pltpu.sync_copy(
        hist_sp.at[pl.ds(sid * BINS_PER_TILE, BINS_PER_TILE)],
        hist_out.at[cid, pl.ds(sid * BINS_PER_TILE, BINS_PER_TILE)],
    )
    oob_v[...] = oob_acc
    pltpu.sync_copy(oob_v, oob_out.at[cid, sid])


@functools.cache
def _build_sc_hist():
    return pl.kernel(
        _sc_body,
        out_type=(
            jax.ShapeDtypeStruct((NUM_CORES, TOTAL_BINS), jnp.float32),
            jax.ShapeDtypeStruct((NUM_CORES, NUM_SUBCORES, VL), jnp.float32),
        ),
        mesh=plsc.VectorSubcoreMesh(
            core_axis_name="c", subcore_axis_name="s",
            num_cores=NUM_CORES, num_subcores=NUM_SUBCORES,
        ),
        scratch_types=[
            pltpu.VMEM((ROWS_PER_W, 128), jnp.float32),
            pltpu.VMEM((ROWS_PER_W, 128), jnp.float32),
            pltpu.VMEM((ROWS_PER_W, 128), jnp.float32),
            pltpu.VMEM((ROWS_PER_W, 128), jnp.float32),
            pltpu.VMEM((ROWS_PER_W, 128), jnp.float32),
            pltpu.VMEM((ROWS_PER_W, 128), jnp.float32),
            pltpu.VMEM((CHUNK,), jnp.float32),
            pltpu.VMEM((CHUNK,), jnp.float32),
            pltpu.VMEM((CHUNK,), jnp.int32),
            pltpu.VMEM((CHUNK,), jnp.int32),
            pltpu.VMEM((CHUNK,), jnp.float32),
            pltpu.VMEM((CHUNK,), jnp.float32),
            pltpu.VMEM((VL,), jnp.float32),
            pltpu.VMEM_SHARED((TOTAL_BINS,), jnp.float32),
            pltpu.SemaphoreType.DMA,
            pltpu.SemaphoreType.DMA,
            pltpu.SemaphoreType.DMA,
            pltpu.SemaphoreType.DMA,
        ],
        compiler_params=pltpu.CompilerParams(needs_layout_passes=False),
    )


def _combine_body(hp_ref, oob_ref, hist_ref, oob_out_ref):
    hist_ref[...] = hp_ref[0] + hp_ref[1]
    oob_out_ref[...] = jnp.sum(oob_ref[...])[None, None]


def kernel(values, weights):
    vals_t = jnp.transpose(values.reshape(NROWS, 128, 3), (2, 0, 1))
    hist_parts, oob_parts = _build_sc_hist()(vals_t, weights)
    hist2, oob11 = pl.pallas_call(
        _combine_body,
        out_shape=(
            jax.ShapeDtypeStruct((TOTAL_BINS // 128, 128), jnp.float32),
            jax.ShapeDtypeStruct((1, 1), jnp.float32),
        ),
    )(hist_parts.reshape(NUM_CORES, TOTAL_BINS // 128, 128),
      oob_parts.reshape(4, 128))
    return hist2.reshape(TOTAL_BINS), oob11[0, 0]
